# Initial kernel scaffold; baseline (speedup 1.0000x reference)
import jax
import jax.numpy as jnp
from jax.experimental import pallas as pl


def kernel(z, w_mat_0, w_pt_0, gamma_0, beta_0, w_mat_1, w_pt_1, gamma_1, beta_1, w_mat_2, w_pt_2, gamma_2, beta_2, w_mat_3, w_pt_3, gamma_3, beta_3, w_mat_4, w_pt_4):
    raise NotImplementedError("write your pallas kernel here")


# speedup vs baseline: 7.4223x; 7.4223x over previous
import jax
import jax.numpy as jnp
from jax.experimental import pallas as pl
from jax.experimental.pallas import tpu as pltpu

BN_EPS = 1e-5
K = 4
N = 2
NZ = 100
C0 = 512


def _bn_relu(ys, gamma, beta, count):
    s = ys[0].sum(axis=0)
    for y in ys[1:]:
        s = s + y.sum(axis=0)
    mean = s / count
    ss = ((ys[0] - mean) ** 2).sum(axis=0)
    for y in ys[1:]:
        ss = ss + ((y - mean) ** 2).sum(axis=0)
    inv = jax.lax.rsqrt(ss / count + BN_EPS)
    return [jnp.maximum((y - mean) * inv * gamma + beta, 0.0) for y in ys]


def _interleave(pars, n, h, w, c):
    a = [p.reshape(n, h, w, c) for p in pars]
    r0 = jnp.stack([a[0], a[1]], axis=3).reshape(n, h, 2 * w, c)
    r1 = jnp.stack([a[2], a[3]], axis=3).reshape(n, h, 2 * w, c)
    return jnp.stack([r0, r1], axis=2).reshape(n, 2 * h, 2 * w, c)


def _up_layer(x, w_ref, xp_ref, h, w, cin, cout):
    xp_ref[...] = jnp.zeros((N, h + 2, w + 2, cin), jnp.float32)
    xp_ref[:, 1:h + 1, 1:w + 1, :] = x
    pars = []
    for di in (0, 1):
        for dj in (0, 1):
            acc = None
            for ky in (di, di + 2):
                for kx in (dj, dj + 2):
                    ay = (di + ky) // 2
                    ax = (dj + kx) // 2
                    slab = xp_ref[:, ay:ay + h, ax:ax + w, :].reshape(
                        N * h * w, cin)
                    t = ky * K + kx
                    wblk = w_ref[t * cin:(t + 1) * cin, :]
                    p = jnp.dot(slab, wblk,
                                preferred_element_type=jnp.float32)
                    acc = p if acc is None else acc + p
            pars.append(acc)
    return pars


def _gen_kernel(z_ref, w0_ref, w1_ref, w2_ref, w3_ref, w4_ref,
                g0_ref, b0_ref, g1_ref, b1_ref, g2_ref, b2_ref,
                g3_ref, b3_ref, out_ref, xp1, xp2, xp3, xp4):
    z = z_ref[...]

    ys = []
    for oy in range(4):
        for ox in range(4):
            t = (3 - oy) * K + (3 - ox)
            wblk = w0_ref[t * NZ:(t + 1) * NZ, :]
            ys.append(jnp.dot(z, wblk, preferred_element_type=jnp.float32))
    y = jnp.stack(ys, axis=1).reshape(N * 16, C0)
    y = _bn_relu([y], g0_ref[...], b0_ref[...], N * 16)[0]
    x = y.reshape(N, 4, 4, C0)

    for w_ref, g_ref, b_ref, xp, h, cin, cout in (
            (w1_ref, g1_ref, b1_ref, xp1, 4, 512, 256),
            (w2_ref, g2_ref, b2_ref, xp2, 8, 256, 128),
            (w3_ref, g3_ref, b3_ref, xp3, 16, 128, 64)):
        pars = _up_layer(x, w_ref, xp, h, h, cin, cout)
        pars = _bn_relu(pars, g_ref[...], b_ref[...], 4 * N * h * h)
        x = _interleave(pars, N, h, h, cout)

    pars = _up_layer(x, w4_ref, xp4, 32, 32, 64, 3)
    pars = [jnp.tanh(p) for p in pars]
    out_ref[...] = _interleave(pars, N, 32, 32, 3)


@jax.jit
def _forward(z2, w0, w1, w2, w3, w4, g0, b0, g1, b1, g2, b2, g3, b3):
    return pl.pallas_call(
        _gen_kernel,
        out_shape=jax.ShapeDtypeStruct((N, 64, 64, 3), jnp.float32),
        scratch_shapes=[
            pltpu.VMEM((N, 6, 6, 512), jnp.float32),
            pltpu.VMEM((N, 10, 10, 256), jnp.float32),
            pltpu.VMEM((N, 18, 18, 128), jnp.float32),
            pltpu.VMEM((N, 34, 34, 64), jnp.float32),
        ],
        compiler_params=pltpu.CompilerParams(
            vmem_limit_bytes=100 * 1024 * 1024),
    )(z2, w0, w1, w2, w3, w4, g0, b0, g1, b1, g2, b2, g3, b3)


def kernel(z, w_mat_0, w_pt_0, gamma_0, beta_0,
           w_mat_1, w_pt_1, gamma_1, beta_1,
           w_mat_2, w_pt_2, gamma_2, beta_2,
           w_mat_3, w_pt_3, gamma_3, beta_3,
           w_mat_4, w_pt_4):
    y = _forward(z.reshape(N, NZ), w_mat_0, w_mat_1, w_mat_2, w_mat_3,
                 w_mat_4, gamma_0, beta_0, gamma_1, beta_1, gamma_2, beta_2,
                 gamma_3, beta_3)
    return jnp.transpose(y, (0, 3, 1, 2))


# speedup vs baseline: 7.5138x; 1.0123x over previous
import jax
import jax.numpy as jnp
from jax.experimental import pallas as pl
from jax.experimental.pallas import tpu as pltpu

BN_EPS = 1e-5
K = 4
N = 2
NZ = 100
C0 = 512


def _bn_relu(ys, gamma, beta, count):
    s = ys[0].sum(axis=0)
    for y in ys[1:]:
        s = s + y.sum(axis=0)
    mean = s / count
    ss = ((ys[0] - mean) ** 2).sum(axis=0)
    for y in ys[1:]:
        ss = ss + ((y - mean) ** 2).sum(axis=0)
    inv = jax.lax.rsqrt(ss / count + BN_EPS)
    return [jnp.maximum((y - mean) * inv * gamma + beta, 0.0) for y in ys]


def _interleave(pars, n, h, w, c):
    a = [p.reshape(n, h, w, c) for p in pars]
    r0 = jnp.stack([a[0], a[1]], axis=3).reshape(n, h, 2 * w, c)
    r1 = jnp.stack([a[2], a[3]], axis=3).reshape(n, h, 2 * w, c)
    return jnp.stack([r0, r1], axis=2).reshape(n, 2 * h, 2 * w, c)


def _up_layer(x, w_ref, xp_ref, h, w, cin, cout):
    xp_ref[:, 0:1, :, :] = jnp.zeros((N, 1, w + 2, cin), jnp.float32)
    xp_ref[:, h + 1:h + 2, :, :] = jnp.zeros((N, 1, w + 2, cin), jnp.float32)
    xp_ref[:, 1:h + 1, 0:1, :] = jnp.zeros((N, h, 1, cin), jnp.float32)
    xp_ref[:, 1:h + 1, w + 1:w + 2, :] = jnp.zeros((N, h, 1, cin),
                                                   jnp.float32)
    xp_ref[:, 1:h + 1, 1:w + 1, :] = x
    pars = []
    for di in (0, 1):
        for dj in (0, 1):
            acc = None
            for ky in (di, di + 2):
                for kx in (dj, dj + 2):
                    ay = (di + ky) // 2
                    ax = (dj + kx) // 2
                    slab = xp_ref[:, ay:ay + h, ax:ax + w, :].reshape(
                        N * h * w, cin)
                    t = ky * K + kx
                    wblk = w_ref[t * cin:(t + 1) * cin, :]
                    p = jnp.dot(slab, wblk,
                                preferred_element_type=jnp.float32)
                    acc = p if acc is None else acc + p
            pars.append(acc)
    return pars


def _gen_kernel(z_ref, w0_hbm, w1_hbm, w2_hbm, w3_hbm, w4_ref,
                g0_ref, b0_ref, g1_ref, b1_ref, g2_ref, b2_ref,
                g3_ref, b3_ref, out_ref,
                w0v, w1v, w2v, w3v, xp1, xp2, xp3, xp4, sems):
    cps = [pltpu.make_async_copy(w0_hbm, w0v, sems.at[0]),
           pltpu.make_async_copy(w1_hbm, w1v, sems.at[1]),
           pltpu.make_async_copy(w2_hbm, w2v, sems.at[2]),
           pltpu.make_async_copy(w3_hbm, w3v, sems.at[3])]
    for cp in cps:
        cp.start()

    z = z_ref[...]

    cps[0].wait()
    ys = []
    for oy in range(4):
        for ox in range(4):
            t = (3 - oy) * K + (3 - ox)
            wblk = w0v[t * NZ:(t + 1) * NZ, :]
            ys.append(jnp.dot(z, wblk, preferred_element_type=jnp.float32))
    y = jnp.stack(ys, axis=1).reshape(N * 16, C0)
    y = _bn_relu([y], g0_ref[...], b0_ref[...], N * 16)[0]
    x = y.reshape(N, 4, 4, C0)

    for cp, w_ref, g_ref, b_ref, xp, h, cin, cout in (
            (cps[1], w1v, g1_ref, b1_ref, xp1, 4, 512, 256),
            (cps[2], w2v, g2_ref, b2_ref, xp2, 8, 256, 128),
            (cps[3], w3v, g3_ref, b3_ref, xp3, 16, 128, 64)):
        cp.wait()
        pars = _up_layer(x, w_ref, xp, h, h, cin, cout)
        pars = _bn_relu(pars, g_ref[...], b_ref[...], 4 * N * h * h)
        x = _interleave(pars, N, h, h, cout)

    pars = _up_layer(x, w4_ref, xp4, 32, 32, 64, 3)
    pars = [jnp.tanh(p) for p in pars]
    out_ref[...] = _interleave(pars, N, 32, 32, 3)


@jax.jit
def _forward(z2, w0, w1, w2, w3, w4, g0, b0, g1, b1, g2, b2, g3, b3):
    vspec = pl.BlockSpec(memory_space=pltpu.MemorySpace.VMEM)
    aspec = pl.BlockSpec(memory_space=pl.ANY)
    return pl.pallas_call(
        _gen_kernel,
        out_shape=jax.ShapeDtypeStruct((N, 64, 64, 3), jnp.float32),
        in_specs=[vspec, aspec, aspec, aspec, aspec, vspec,
                  vspec, vspec, vspec, vspec, vspec, vspec, vspec, vspec],
        out_specs=vspec,
        scratch_shapes=[
            pltpu.VMEM((16 * NZ, 512), jnp.float32),
            pltpu.VMEM((16 * 512, 256), jnp.float32),
            pltpu.VMEM((16 * 256, 128), jnp.float32),
            pltpu.VMEM((16 * 128, 64), jnp.float32),
            pltpu.VMEM((N, 6, 6, 512), jnp.float32),
            pltpu.VMEM((N, 10, 10, 256), jnp.float32),
            pltpu.VMEM((N, 18, 18, 128), jnp.float32),
            pltpu.VMEM((N, 34, 34, 64), jnp.float32),
            pltpu.SemaphoreType.DMA((4,)),
        ],
        compiler_params=pltpu.CompilerParams(
            vmem_limit_bytes=100 * 1024 * 1024),
    )(z2, w0, w1, w2, w3, w4, g0, b0, g1, b1, g2, b2, g3, b3)


def kernel(z, w_mat_0, w_pt_0, gamma_0, beta_0,
           w_mat_1, w_pt_1, gamma_1, beta_1,
           w_mat_2, w_pt_2, gamma_2, beta_2,
           w_mat_3, w_pt_3, gamma_3, beta_3,
           w_mat_4, w_pt_4):
    y = _forward(z.reshape(N, NZ), w_mat_0, w_mat_1, w_mat_2, w_mat_3,
                 w_mat_4, gamma_0, beta_0, gamma_1, beta_1, gamma_2, beta_2,
                 gamma_3, beta_3)
    return jnp.transpose(y, (0, 3, 1, 2))


# speedup vs baseline: 9.0176x; 1.2001x over previous
import jax
import jax.numpy as jnp
from jax.experimental import pallas as pl
from jax.experimental.pallas import tpu as pltpu

BN_EPS = 1e-5
K = 4
N = 2
NZ = 100
C0 = 512


def _bn_relu(ys, gamma, beta, count):
    s = ys[0].sum(axis=0)
    for y in ys[1:]:
        s = s + y.sum(axis=0)
    mean = s / count
    ss = ((ys[0] - mean) ** 2).sum(axis=0)
    for y in ys[1:]:
        ss = ss + ((y - mean) ** 2).sum(axis=0)
    inv = jax.lax.rsqrt(ss / count + BN_EPS)
    return [jnp.maximum((y - mean) * inv * gamma + beta, 0.0) for y in ys]


def _interleave(pars, n, h, w, c):
    a = [p.reshape(n, h, w, c) for p in pars]
    r0 = jnp.stack([a[0], a[1]], axis=3).reshape(n, h, 2 * w, c)
    r1 = jnp.stack([a[2], a[3]], axis=3).reshape(n, h, 2 * w, c)
    return jnp.stack([r0, r1], axis=2).reshape(n, 2 * h, 2 * w, c)


def _up_layer(x, w_ref, xp_ref, h, w, cin, cout):
    xp_ref[:, 0:1, :, :] = jnp.zeros((N, 1, w + 2, cin), jnp.float32)
    xp_ref[:, h + 1:h + 2, :, :] = jnp.zeros((N, 1, w + 2, cin), jnp.float32)
    xp_ref[:, 1:h + 1, 0:1, :] = jnp.zeros((N, h, 1, cin), jnp.float32)
    xp_ref[:, 1:h + 1, w + 1:w + 2, :] = jnp.zeros((N, h, 1, cin),
                                                   jnp.float32)
    xp_ref[:, 1:h + 1, 1:w + 1, :] = x
    pars = []
    for di in (0, 1):
        for dj in (0, 1):
            acc = None
            for ky in (di, di + 2):
                for kx in (dj, dj + 2):
                    ay = (di + ky) // 2
                    ax = (dj + kx) // 2
                    slab = xp_ref[:, ay:ay + h, ax:ax + w, :].reshape(
                        N * h * w, cin)
                    t = ky * K + kx
                    wblk = w_ref[t * cin:(t + 1) * cin, :]
                    p = jnp.dot(slab, wblk,
                                preferred_element_type=jnp.float32)
                    acc = p if acc is None else acc + p
            pars.append(acc)
    return pars


def _gen_kernel(z_ref, w0_hbm, w1_hbm, w2_hbm, w3_hbm, w4_ref,
                g0_ref, b0_ref, g1_ref, b1_ref, g2_ref, b2_ref,
                g3_ref, b3_ref, out_ref,
                w0v, w1v, w2v, w3v, xp1, xp2, xp3, xp4, sems):
    cps = [pltpu.make_async_copy(w0_hbm, w0v, sems.at[0]),
           pltpu.make_async_copy(w1_hbm, w1v, sems.at[1]),
           pltpu.make_async_copy(w2_hbm, w2v, sems.at[2]),
           pltpu.make_async_copy(w3_hbm, w3v, sems.at[3])]
    for cp in cps:
        cp.start()

    z = z_ref[...].reshape(N, NZ)

    cps[0].wait()
    ys = []
    for oy in range(4):
        for ox in range(4):
            t = (3 - oy) * K + (3 - ox)
            wblk = w0v[t * NZ:(t + 1) * NZ, :]
            ys.append(jnp.dot(z, wblk, preferred_element_type=jnp.float32))
    y = jnp.stack(ys, axis=1).reshape(N * 16, C0)
    y = _bn_relu([y], g0_ref[...], b0_ref[...], N * 16)[0]
    x = y.reshape(N, 4, 4, C0)

    for cp, w_ref, g_ref, b_ref, xp, h, cin, cout in (
            (cps[1], w1v, g1_ref, b1_ref, xp1, 4, 512, 256),
            (cps[2], w2v, g2_ref, b2_ref, xp2, 8, 256, 128),
            (cps[3], w3v, g3_ref, b3_ref, xp3, 16, 128, 64)):
        cp.wait()
        pars = _up_layer(x, w_ref, xp, h, h, cin, cout)
        pars = _bn_relu(pars, g_ref[...], b_ref[...], 4 * N * h * h)
        x = _interleave(pars, N, h, h, cout)

    pars = _up_layer(x, w4_ref, xp4, 32, 32, 64, 3)
    pars = [jnp.tanh(p) for p in pars]
    y = _interleave(pars, N, 32, 32, 3)
    out_ref[...] = jnp.transpose(y, (0, 3, 1, 2))


@jax.jit
def _forward(z2, w0, w1, w2, w3, w4, g0, b0, g1, b1, g2, b2, g3, b3):
    vspec = pl.BlockSpec(memory_space=pltpu.MemorySpace.VMEM)
    aspec = pl.BlockSpec(memory_space=pl.ANY)
    return pl.pallas_call(
        _gen_kernel,
        out_shape=jax.ShapeDtypeStruct((N, 3, 64, 64), jnp.float32),
        in_specs=[vspec, aspec, aspec, aspec, aspec, vspec,
                  vspec, vspec, vspec, vspec, vspec, vspec, vspec, vspec],
        out_specs=vspec,
        scratch_shapes=[
            pltpu.VMEM((16 * NZ, 512), jnp.float32),
            pltpu.VMEM((16 * 512, 256), jnp.float32),
            pltpu.VMEM((16 * 256, 128), jnp.float32),
            pltpu.VMEM((16 * 128, 64), jnp.float32),
            pltpu.VMEM((N, 6, 6, 512), jnp.float32),
            pltpu.VMEM((N, 10, 10, 256), jnp.float32),
            pltpu.VMEM((N, 18, 18, 128), jnp.float32),
            pltpu.VMEM((N, 34, 34, 64), jnp.float32),
            pltpu.SemaphoreType.DMA((4,)),
        ],
        compiler_params=pltpu.CompilerParams(
            vmem_limit_bytes=100 * 1024 * 1024),
    )(z2, w0, w1, w2, w3, w4, g0, b0, g1, b1, g2, b2, g3, b3)


def kernel(z, w_mat_0, w_pt_0, gamma_0, beta_0,
           w_mat_1, w_pt_1, gamma_1, beta_1,
           w_mat_2, w_pt_2, gamma_2, beta_2,
           w_mat_3, w_pt_3, gamma_3, beta_3,
           w_mat_4, w_pt_4):
    return _forward(z, w_mat_0, w_mat_1, w_mat_2, w_mat_3,
                    w_mat_4, gamma_0, beta_0, gamma_1, beta_1, gamma_2,
                    beta_2, gamma_3, beta_3)
